# baseline (speedup 1.0000x reference)
import jax
import jax.numpy as jnp
from jax.experimental import pallas as pl
from jax.experimental.pallas import tpu as pltpu

_MAX_ITER = 10
_EPSILON = 1e-4


def _sinkhorn_body(s_ref, o_ref):
    s = s_ref[0]
    nonzero_mask = (s != 0.0).astype(s.dtype)
    for i in range(_MAX_ITER):
        if i % 2 == 0:
            col_sum = jnp.sum(s, axis=0, keepdims=True)
            s = s * (1.0 / col_sum) * nonzero_mask
        else:
            row_sum = jnp.sum(s, axis=1, keepdims=True)
            s = (1.0 / (row_sum + _EPSILON)) * s * nonzero_mask
    o_ref[0] = s


def kernel(s):
    b, n1, n2 = s.shape
    return pl.pallas_call(
        _sinkhorn_body,
        grid=(b,),
        in_specs=[pl.BlockSpec((1, n1, n2), lambda i: (i, 0, 0))],
        out_specs=pl.BlockSpec((1, n1, n2), lambda i: (i, 0, 0)),
        out_shape=jax.ShapeDtypeStruct(s.shape, s.dtype),
        compiler_params=pltpu.CompilerParams(
            dimension_semantics=("parallel",),
        ),
    )(s)


# speedup vs baseline: 1.1379x; 1.1379x over previous
import jax
import jax.numpy as jnp
from jax.experimental import pallas as pl
from jax.experimental.pallas import tpu as pltpu

_EPSILON = 1e-4
_N_PAIRS = 4


def _sinkhorn_body(s_ref, o_ref):
    s0 = s_ref[0]
    m = jnp.sum(s0, axis=0, keepdims=True)
    v = 1.0 / m
    r = jnp.sum(s0 * v, axis=1, keepdims=True)
    u = 1.0 / (r + _EPSILON)
    for _ in range(_N_PAIRS):
        m = jnp.sum(s0 * u, axis=0, keepdims=True)
        v = 1.0 / m
        r = jnp.sum(s0 * v, axis=1, keepdims=True)
        u = u / (u * r + _EPSILON)
    o_ref[0] = s0 * u * v


def kernel(s):
    b, n1, n2 = s.shape
    return pl.pallas_call(
        _sinkhorn_body,
        grid=(b,),
        in_specs=[pl.BlockSpec((1, n1, n2), lambda i: (i, 0, 0))],
        out_specs=pl.BlockSpec((1, n1, n2), lambda i: (i, 0, 0)),
        out_shape=jax.ShapeDtypeStruct(s.shape, s.dtype),
        compiler_params=pltpu.CompilerParams(
            dimension_semantics=("parallel",),
        ),
    )(s)


# speedup vs baseline: 1.5277x; 1.3426x over previous
import jax
import jax.numpy as jnp
from jax.experimental import pallas as pl
from jax.experimental.pallas import tpu as pltpu

_EPSILON = 1e-4
_N_PAIRS = 4


def _sinkhorn_body(s_ref, o_ref):
    for j in range(s_ref.shape[0]):
        _sinkhorn_one(s_ref, o_ref, j)


def _sinkhorn_one(s_ref, o_ref, j):
    s0 = s_ref[j]
    m = jnp.sum(s0, axis=0, keepdims=True)
    v = 1.0 / m
    r = jnp.sum(s0 * v, axis=1, keepdims=True)
    u = 1.0 / (r + _EPSILON)
    for _ in range(_N_PAIRS):
        m = jnp.sum(s0 * u, axis=0, keepdims=True)
        v = 1.0 / m
        r = jnp.sum(s0 * v, axis=1, keepdims=True)
        u = u / (u * r + _EPSILON)
    o_ref[j] = s0 * u * v


def kernel(s):
    b, n1, n2 = s.shape
    return pl.pallas_call(
        _sinkhorn_body,
        grid=(b // 2,),
        in_specs=[pl.BlockSpec((2, n1, n2), lambda i: (i, 0, 0))],
        out_specs=pl.BlockSpec((2, n1, n2), lambda i: (i, 0, 0)),
        out_shape=jax.ShapeDtypeStruct(s.shape, s.dtype),
        compiler_params=pltpu.CompilerParams(
            dimension_semantics=("parallel",),
        ),
    )(s)


# speedup vs baseline: 1.7520x; 1.1468x over previous
import jax
import jax.numpy as jnp
from jax.experimental import pallas as pl
from jax.experimental.pallas import tpu as pltpu

_EPSILON = 1e-4
_N_PAIRS = 4


def _sinkhorn_body(s_ref, o_ref):
    for j in range(s_ref.shape[0]):
        _sinkhorn_one(s_ref, o_ref, j)


def _sinkhorn_one(s_ref, o_ref, j):
    s0 = s_ref[j]
    m = jnp.sum(s0, axis=0, keepdims=True)
    v = 1.0 / m
    r = jnp.sum(s0 * v, axis=1, keepdims=True)
    u = 1.0 / (r + _EPSILON)
    for _ in range(_N_PAIRS):
        m = jnp.sum(s0 * u, axis=0, keepdims=True)
        v = 1.0 / m
        r = jnp.sum(s0 * v, axis=1, keepdims=True)
        u = u / (u * r + _EPSILON)
    o_ref[j] = s0 * u * v


def kernel(s):
    b, n1, n2 = s.shape
    return pl.pallas_call(
        _sinkhorn_body,
        grid=(b // 4,),
        in_specs=[pl.BlockSpec((4, n1, n2), lambda i: (i, 0, 0))],
        out_specs=pl.BlockSpec((4, n1, n2), lambda i: (i, 0, 0)),
        out_shape=jax.ShapeDtypeStruct(s.shape, s.dtype),
        compiler_params=pltpu.CompilerParams(
            dimension_semantics=("parallel",),
        ),
    )(s)
